# baseline (speedup 1.0000x reference)
import functools
import math

import jax
import jax.numpy as jnp
import numpy as np
from jax import lax
from jax.experimental import pallas as pl
from jax.experimental.pallas import tpu as pltpu
from jax.experimental.pallas import tpu_sc as plsc

_LANES = 16


def _pe_flat(batch: int, seq: int, d: int) -> np.ndarray:
    position = np.arange(seq, dtype=np.float32)[:, None]
    div_term = np.exp(
        np.arange(0, d, 2, dtype=np.float32) * (-math.log(10000.0) / d)
    )
    pe = np.zeros((seq, d), dtype=np.float32)
    pe[:, 0::2] = np.sin(position * div_term)
    pe[:, 1::2] = np.cos(position * div_term)
    return np.tile(pe, (batch, 1))


def kernel(x, table):
    batch, seq = x.shape
    _, d = table.shape
    b_total = batch * seq
    scale = math.sqrt(d)

    info = plsc.get_sparse_core_info()
    nc, ns = info.num_cores, info.num_subcores
    nw = nc * ns
    b_per_w = b_total // nw
    assert b_total % (8 * nw) == 0 and d % _LANES == 0

    pe = jnp.asarray(_pe_flat(batch, seq, d))
    idx = x.reshape(-1).astype(jnp.int32)

    mesh = plsc.VectorSubcoreMesh(core_axis_name="c", subcore_axis_name="s")

    @functools.partial(
        pl.kernel,
        mesh=mesh,
        out_type=jax.ShapeDtypeStruct((b_total, d), jnp.float32),
        scratch_types=[
            pltpu.VMEM((b_per_w,), jnp.int32),
            pltpu.VMEM((b_per_w, d), jnp.float32),
            pltpu.VMEM((b_per_w, d), jnp.float32),
            pltpu.SemaphoreType.DMA,
        ],
    )
    def emb_kernel(idx_hbm, table_hbm, pe_hbm, out_hbm, idx_v, rows_v, pe_v, sem):
        wid = lax.axis_index("s") * nc + lax.axis_index("c")
        base = wid * b_per_w
        pltpu.sync_copy(idx_hbm.at[pl.ds(base, b_per_w)], idx_v)
        gather = pltpu.async_copy(table_hbm.at[idx_v], rows_v, sem)
        pltpu.sync_copy(pe_hbm.at[pl.ds(base, b_per_w)], pe_v)
        gather.wait()

        def body(i, carry):
            for j in range(d // _LANES):
                sl = pl.ds(j * _LANES, _LANES)
                rows_v[i, sl] = rows_v[i, sl] * scale + pe_v[i, sl]
            return carry

        lax.fori_loop(0, b_per_w, body, 0)
        pltpu.sync_copy(rows_v, out_hbm.at[pl.ds(base, b_per_w)])

    out = emb_kernel(idx, table, pe)
    return out.reshape(batch, seq, d)


# speedup vs baseline: 1.1025x; 1.1025x over previous
import functools
import math

import jax
import jax.numpy as jnp
import numpy as np
from jax import lax
from jax.experimental import pallas as pl
from jax.experimental.pallas import tpu as pltpu
from jax.experimental.pallas import tpu_sc as plsc

_LANES = 16
_NCHUNK = 4


def _pe_table(seq: int, d: int) -> np.ndarray:
    position = np.arange(seq, dtype=np.float32)[:, None]
    div_term = np.exp(
        np.arange(0, d, 2, dtype=np.float32) * (-math.log(10000.0) / d)
    )
    pe = np.zeros((seq, d), dtype=np.float32)
    pe[:, 0::2] = np.sin(position * div_term)
    pe[:, 1::2] = np.cos(position * div_term)
    return pe


def kernel(x, table):
    batch, seq = x.shape
    _, d = table.shape
    b_total = batch * seq
    scale = math.sqrt(d)

    info = plsc.get_sparse_core_info()
    nc, ns = info.num_cores, info.num_subcores
    nw = nc * ns
    b_per_w = b_total // nw
    chunk = b_per_w // _NCHUNK
    assert b_total % (8 * nw) == 0 and d % _LANES == 0
    assert seq % b_per_w == 0 and chunk % 8 == 0

    pe_flat = jnp.asarray(_pe_table(seq, d).reshape(-1))
    idx = x.reshape(-1).astype(jnp.int32)

    mesh = plsc.VectorSubcoreMesh(core_axis_name="c", subcore_axis_name="s")

    @functools.partial(
        pl.kernel,
        mesh=mesh,
        out_type=jax.ShapeDtypeStruct((b_total, d), jnp.float32),
        scratch_types=[
            pltpu.VMEM((b_per_w,), jnp.int32),
            pltpu.VMEM((b_per_w, d), jnp.float32),
            pltpu.VMEM((b_per_w * d,), jnp.float32),
            pltpu.SemaphoreType.DMA,
        ]
        + [pltpu.SemaphoreType.DMA for _ in range(_NCHUNK)]
        + [pltpu.SemaphoreType.DMA],
    )
    def emb_kernel(
        idx_hbm, table_hbm, pe_hbm, out_hbm, idx_v, rows_v, pe_v, pe_sem,
        *sems,
    ):
        gather_sems, st_sem = sems[:_NCHUNK], sems[_NCHUNK]
        wid = lax.axis_index("s") * nc + lax.axis_index("c")
        base = wid * b_per_w
        t0 = lax.rem(base, seq)

        pe_cp = pltpu.async_copy(
            pe_hbm.at[pl.ds(t0 * d, b_per_w * d)], pe_v, pe_sem
        )
        pltpu.sync_copy(idx_hbm.at[pl.ds(base, b_per_w)], idx_v)
        gathers = []
        for k in range(_NCHUNK):
            gathers.append(
                pltpu.async_copy(
                    table_hbm.at[idx_v.at[pl.ds(k * chunk, chunk)]],
                    rows_v.at[pl.ds(k * chunk, chunk)],
                    gather_sems[k],
                )
            )
        pe_cp.wait()

        stores = []
        for k in range(_NCHUNK):
            gathers[k].wait()

            def body(i, carry):
                for j in range(d // _LANES):
                    sl = pl.ds(j * _LANES, _LANES)
                    rows_v[i, sl] = (
                        rows_v[i, sl] * scale
                        + pe_v[pl.ds(i * d + j * _LANES, _LANES)]
                    )
                return carry

            lax.fori_loop(k * chunk, (k + 1) * chunk, body, 0)
            stores.append(
                pltpu.async_copy(
                    rows_v.at[pl.ds(k * chunk, chunk)],
                    out_hbm.at[pl.ds(base + k * chunk, chunk)],
                    st_sem,
                )
            )
        for st in stores:
            st.wait()

    out = emb_kernel(idx, table, pe_flat)
    return out.reshape(batch, seq, d)
